# Initial kernel scaffold; baseline (speedup 1.0000x reference)
import jax
import jax.numpy as jnp
from jax import lax
from jax.experimental import pallas as pl
from jax.experimental.pallas import tpu as pltpu
from jax.experimental.pallas import tpu_sc as plsc

N = 10000
E = 320000
D = 128

NC = 2
NS = 16
NW = NC * NS
EPW = E // NW
BATCH = 80
NB = EPW // BATCH
N_PAD = 10240
RPZ = N_PAD // NS

_mesh = plsc.VectorSubcoreMesh(core_axis_name="c", subcore_axis_name="s")


def _deg_body(dst_hbm, ew_hbm, zero_hbm, out_hbm, dst_v, ew_v, shared_deg):
    c = lax.axis_index("c")
    s = lax.axis_index("s")
    row0 = pl.multiple_of(s * RPZ, 8)
    pltpu.sync_copy(dst_hbm.at[c, s], dst_v)
    pltpu.sync_copy(ew_hbm.at[c, s], ew_v)
    pltpu.sync_copy(zero_hbm.at[pl.ds(row0, RPZ)], shared_deg.at[pl.ds(row0, RPZ)])
    plsc.subcore_barrier()

    def body(j, carry):
        pltpu.sync_copy(ew_v.at[j], shared_deg.at[dst_v.at[j]], add=True)
        return carry

    lax.fori_loop(0, NB, body, 0)
    plsc.subcore_barrier()
    pltpu.sync_copy(shared_deg.at[pl.ds(row0, RPZ)], out_hbm.at[c, pl.ds(row0, RPZ)])


_deg_sc = pl.kernel(
    _deg_body,
    out_type=jax.ShapeDtypeStruct((NC, N_PAD), jnp.float32),
    mesh=_mesh,
    scratch_types=[
        pltpu.VMEM((NB, BATCH), jnp.int32),
        pltpu.VMEM((NB, BATCH), jnp.float32),
        pltpu.VMEM_SHARED((N_PAD,), jnp.float32),
    ],
)


def _agg_body(h_hbm, src_hbm, dst_hbm, ew_hbm, zero_hbm, out_hbm,
              src_v, dst_v, ew_v, rows, shared, gsem):
    c = lax.axis_index("c")
    s = lax.axis_index("s")
    row0 = pl.multiple_of(s * RPZ, 8)
    pltpu.sync_copy(src_hbm.at[c, s], src_v)
    pltpu.sync_copy(dst_hbm.at[c, s], dst_v)
    pltpu.sync_copy(ew_hbm.at[c, s], ew_v)
    pltpu.sync_copy(zero_hbm.at[pl.ds(row0, RPZ)], shared.at[pl.ds(row0, RPZ)])
    plsc.subcore_barrier()

    def jbody(j, carry):
        e0 = pl.multiple_of(j * BATCH, 8)
        pltpu.async_copy(h_hbm.at[src_v.at[pl.ds(e0, BATCH)]], rows, gsem).wait()

        @plsc.parallel_loop(0, BATCH, 1, unroll=4)
        def ebody(e):
            w = plsc.load_gather(ew_v, [jnp.full((16,), e0 + e, jnp.int32)])
            for k in range(D // 16):
                sl = pl.ds(k * 16, 16)
                rows[e, sl] = rows[e, sl] * w

        pltpu.sync_copy(rows, shared.at[dst_v.at[j]], add=True)
        return carry

    lax.fori_loop(0, NB, jbody, 0)
    plsc.subcore_barrier()
    pltpu.sync_copy(shared.at[pl.ds(row0, RPZ)], out_hbm.at[c, pl.ds(row0, RPZ)])


_agg_sc = pl.kernel(
    _agg_body,
    out_type=jax.ShapeDtypeStruct((NC, N_PAD, D), jnp.float32),
    mesh=_mesh,
    scratch_types=[
        pltpu.VMEM((EPW,), jnp.int32),
        pltpu.VMEM((NB, BATCH), jnp.int32),
        pltpu.VMEM((EPW,), jnp.float32),
        pltpu.VMEM((BATCH, D), jnp.float32),
        pltpu.VMEM_SHARED((N_PAD, D), jnp.float32),
        pltpu.SemaphoreType.DMA,
    ],
)


BR = 2000


def _tc1_body(d0_ref, d1_ref, x_ref, w_ref, h1p_ref, dis_ref):
    deg = d0_ref[...] + d1_ref[...] + 1.0
    dis = jnp.where(deg > 0, lax.rsqrt(deg), 0.0)
    h = jnp.dot(x_ref[...], w_ref[...], preferred_element_type=jnp.float32)
    h1p_ref[...] = dis * h
    dis_ref[...] = dis


def _tc2_body(s0_ref, s1_ref, hp_ref, dis_ref, b_ref, w_ref, out_ref):
    dis = dis_ref[...]
    z = dis * (s0_ref[...] + s1_ref[...] + hp_ref[...]) + b_ref[...]
    z = jnp.maximum(z, 0.0)
    out_ref[...] = dis * jnp.dot(z, w_ref[...], preferred_element_type=jnp.float32)


def _tc3_body(s0_ref, s1_ref, hp_ref, dis_ref, b_ref, g_ref, be_ref, out_ref):
    a = dis_ref[...] * (s0_ref[...] + s1_ref[...] + hp_ref[...]) + b_ref[...]
    m = jnp.mean(a, axis=1, keepdims=True)
    v = jnp.mean((a - m) ** 2, axis=1, keepdims=True)
    out_ref[...] = (a - m) * lax.rsqrt(v + 1e-5) * g_ref[...] + be_ref[...]


_rows_spec = pl.BlockSpec((BR, D), lambda i: (i, 0))
_col_spec = pl.BlockSpec((BR, 1), lambda i: (i, 0))
_w_spec = pl.BlockSpec((D, D), lambda i: (0, 0))
_vec_spec = pl.BlockSpec((1, D), lambda i: (0, 0))

_tc1 = pl.pallas_call(
    _tc1_body,
    grid=(N // BR,),
    in_specs=[_col_spec, _col_spec, _rows_spec, _w_spec],
    out_specs=[_rows_spec, _col_spec],
    out_shape=[
        jax.ShapeDtypeStruct((N, D), jnp.float32),
        jax.ShapeDtypeStruct((N, 1), jnp.float32),
    ],
)

_tc2 = pl.pallas_call(
    _tc2_body,
    grid=(N // BR,),
    in_specs=[_rows_spec, _rows_spec, _rows_spec, _col_spec, _vec_spec, _w_spec],
    out_specs=_rows_spec,
    out_shape=jax.ShapeDtypeStruct((N, D), jnp.float32),
)

_tc3 = pl.pallas_call(
    _tc3_body,
    grid=(N // BR,),
    in_specs=[_rows_spec, _rows_spec, _rows_spec, _col_spec, _vec_spec,
              _vec_spec, _vec_spec],
    out_specs=_rows_spec,
    out_shape=jax.ShapeDtypeStruct((N, D), jnp.float32),
)


@jax.jit
def kernel(x, edge_index, edge_weight, W1, b1, W2, b2, gamma, beta):
    src = edge_index[0].astype(jnp.int32)
    dst = edge_index[1].astype(jnp.int32)
    ew = edge_weight.astype(jnp.float32)

    src_r = src.reshape(NC, NS, EPW)
    dst_r = dst.reshape(NC, NS, NB, BATCH)
    ew_r = ew.reshape(NC, NS, EPW)
    ew_b = ew.reshape(NC, NS, NB, BATCH)
    zero_col = jnp.zeros((N_PAD,), jnp.float32)
    zero_rows = jnp.zeros((N_PAD, D), jnp.float32)

    deg_p = _deg_sc(dst_r, ew_b, zero_col)
    d0 = deg_p[0, :N].reshape(N, 1)
    d1 = deg_p[1, :N].reshape(N, 1)
    h1p, dis = _tc1(d0, d1, x, W1)

    s1 = _agg_sc(h1p, src_r, dst_r, ew_r, zero_rows)
    h2p = _tc2(s1[0, :N], s1[1, :N], h1p, dis, b1.reshape(1, D), W2)

    s2 = _agg_sc(h2p, src_r, dst_r, ew_r, zero_rows)
    out = _tc3(s2[0, :N], s2[1, :N], h2p, dis, b2.reshape(1, D),
               gamma.reshape(1, D), beta.reshape(1, D))
    return out


# speedup vs baseline: 16.3112x; 16.3112x over previous
import jax
import jax.numpy as jnp
from jax import lax
from jax.experimental import pallas as pl
from jax.experimental.pallas import tpu as pltpu
from jax.experimental.pallas import tpu_sc as plsc

N = 10000
E = 320000
D = 128

NC = 2
NS = 16
NW = NC * NS
EPW = E // NW
BATCH = 80
NB = EPW // BATCH
N_PAD = 10240
RPZ = N_PAD // NS

_mesh = plsc.VectorSubcoreMesh(core_axis_name="c", subcore_axis_name="s")


def _deg_body(dst_hbm, ew_hbm, zero_hbm, out_hbm, dst_v, ew_v, shared_deg):
    c = lax.axis_index("c")
    s = lax.axis_index("s")
    row0 = pl.multiple_of(s * RPZ, 8)
    pltpu.sync_copy(dst_hbm.at[c, s], dst_v)
    pltpu.sync_copy(ew_hbm.at[c, s], ew_v)
    pltpu.sync_copy(zero_hbm.at[pl.ds(row0, RPZ)], shared_deg.at[pl.ds(row0, RPZ)])
    plsc.subcore_barrier()

    def body(j, carry):
        pltpu.sync_copy(ew_v.at[j], shared_deg.at[dst_v.at[j]], add=True)
        return carry

    lax.fori_loop(0, NB, body, 0)
    plsc.subcore_barrier()
    pltpu.sync_copy(shared_deg.at[pl.ds(row0, RPZ)], out_hbm.at[c, pl.ds(row0, RPZ)])


_deg_sc = pl.kernel(
    _deg_body,
    out_type=jax.ShapeDtypeStruct((NC, N_PAD), jnp.float32),
    mesh=_mesh,
    scratch_types=[
        pltpu.VMEM((NB, BATCH), jnp.int32),
        pltpu.VMEM((NB, BATCH), jnp.float32),
        pltpu.VMEM_SHARED((N_PAD,), jnp.float32),
    ],
    compiler_params=pltpu.CompilerParams(needs_layout_passes=False),
)


def _agg_body(h_hbm, src_hbm, dst_hbm, ew_hbm, zero_hbm, out_hbm,
              src_v, dst_v, ew_v, rows, shared, gsem):
    c = lax.axis_index("c")
    s = lax.axis_index("s")
    row0 = pl.multiple_of(s * RPZ, 8)
    pltpu.sync_copy(src_hbm.at[c, s], src_v)
    pltpu.sync_copy(dst_hbm.at[c, s], dst_v)
    pltpu.sync_copy(ew_hbm.at[c, s], ew_v)
    pltpu.sync_copy(zero_hbm.at[pl.ds(row0, RPZ)], shared.at[pl.ds(row0, RPZ)])
    plsc.subcore_barrier()

    def jbody(j, carry):
        e0 = pl.multiple_of(j * BATCH, 8)
        pltpu.async_copy(h_hbm.at[src_v.at[pl.ds(e0, BATCH)]], rows, gsem).wait()

        @plsc.parallel_loop(0, BATCH, 1, unroll=4)
        def ebody(e):
            w = plsc.load_gather(ew_v, [jnp.full((16,), e0 + e, jnp.int32)])
            for k in range(D // 16):
                sl = pl.ds(k * 16, 16)
                rows[e, sl] = rows[e, sl] * w

        pltpu.sync_copy(rows, shared.at[dst_v.at[j]], add=True)
        return carry

    lax.fori_loop(0, NB, jbody, 0)
    plsc.subcore_barrier()
    pltpu.sync_copy(shared.at[pl.ds(row0, RPZ)], out_hbm.at[c, pl.ds(row0, RPZ)])


_agg_sc = pl.kernel(
    _agg_body,
    out_type=jax.ShapeDtypeStruct((NC, N_PAD, D), jnp.float32),
    mesh=_mesh,
    scratch_types=[
        pltpu.VMEM((EPW,), jnp.int32),
        pltpu.VMEM((NB, BATCH), jnp.int32),
        pltpu.VMEM((EPW,), jnp.float32),
        pltpu.VMEM((BATCH, D), jnp.float32),
        pltpu.VMEM_SHARED((N_PAD, D), jnp.float32),
        pltpu.SemaphoreType.DMA,
    ],
    compiler_params=pltpu.CompilerParams(needs_layout_passes=False),
)


BR = 2000


def _tc1_body(d0_ref, d1_ref, x_ref, w_ref, h1p_ref, dis_ref):
    deg = d0_ref[...] + d1_ref[...] + 1.0
    dis = jnp.where(deg > 0, lax.rsqrt(deg), 0.0)
    h = jnp.dot(x_ref[...], w_ref[...], preferred_element_type=jnp.float32)
    h1p_ref[...] = dis * h
    dis_ref[...] = dis


def _tc2_body(s0_ref, s1_ref, hp_ref, dis_ref, b_ref, w_ref, out_ref):
    dis = dis_ref[...]
    z = dis * (s0_ref[...] + s1_ref[...] + hp_ref[...]) + b_ref[...]
    z = jnp.maximum(z, 0.0)
    out_ref[...] = dis * jnp.dot(z, w_ref[...], preferred_element_type=jnp.float32)


def _tc3_body(s0_ref, s1_ref, hp_ref, dis_ref, b_ref, g_ref, be_ref, out_ref):
    a = dis_ref[...] * (s0_ref[...] + s1_ref[...] + hp_ref[...]) + b_ref[...]
    m = jnp.mean(a, axis=1, keepdims=True)
    v = jnp.mean((a - m) ** 2, axis=1, keepdims=True)
    out_ref[...] = (a - m) * lax.rsqrt(v + 1e-5) * g_ref[...] + be_ref[...]


_rows_spec = pl.BlockSpec((BR, D), lambda i: (i, 0))
_col_spec = pl.BlockSpec((BR, 1), lambda i: (i, 0))
_w_spec = pl.BlockSpec((D, D), lambda i: (0, 0))
_vec_spec = pl.BlockSpec((1, D), lambda i: (0, 0))

_tc1 = pl.pallas_call(
    _tc1_body,
    grid=(N // BR,),
    in_specs=[_col_spec, _col_spec, _rows_spec, _w_spec],
    out_specs=[_rows_spec, _col_spec],
    out_shape=[
        jax.ShapeDtypeStruct((N, D), jnp.float32),
        jax.ShapeDtypeStruct((N, 1), jnp.float32),
    ],
)

_tc2 = pl.pallas_call(
    _tc2_body,
    grid=(N // BR,),
    in_specs=[_rows_spec, _rows_spec, _rows_spec, _col_spec, _vec_spec, _w_spec],
    out_specs=_rows_spec,
    out_shape=jax.ShapeDtypeStruct((N, D), jnp.float32),
)

_tc3 = pl.pallas_call(
    _tc3_body,
    grid=(N // BR,),
    in_specs=[_rows_spec, _rows_spec, _rows_spec, _col_spec, _vec_spec,
              _vec_spec, _vec_spec],
    out_specs=_rows_spec,
    out_shape=jax.ShapeDtypeStruct((N, D), jnp.float32),
)


@jax.jit
def kernel(x, edge_index, edge_weight, W1, b1, W2, b2, gamma, beta):
    src = edge_index[0].astype(jnp.int32)
    dst = edge_index[1].astype(jnp.int32)
    ew = edge_weight.astype(jnp.float32)

    src_r = src.reshape(NC, NS, EPW)
    dst_r = dst.reshape(NC, NS, NB, BATCH)
    ew_r = ew.reshape(NC, NS, EPW)
    ew_b = ew.reshape(NC, NS, NB, BATCH)
    zero_col = jnp.zeros((N_PAD,), jnp.float32)
    zero_rows = jnp.zeros((N_PAD, D), jnp.float32)

    deg_p = _deg_sc(dst_r, ew_b, zero_col)
    d0 = deg_p[0, :N].reshape(N, 1)
    d1 = deg_p[1, :N].reshape(N, 1)
    h1p, dis = _tc1(d0, d1, x, W1)

    s1 = _agg_sc(h1p, src_r, dst_r, ew_r, zero_rows)
    h2p = _tc2(s1[0, :N], s1[1, :N], h1p, dis, b1.reshape(1, D), W2)

    s2 = _agg_sc(h2p, src_r, dst_r, ew_r, zero_rows)
    out = _tc3(s2[0, :N], s2[1, :N], h2p, dis, b2.reshape(1, D),
               gamma.reshape(1, D), beta.reshape(1, D))
    return out


# speedup vs baseline: 22.9154x; 1.4049x over previous
import jax
import jax.numpy as jnp
from jax import lax
from jax.experimental import pallas as pl
from jax.experimental.pallas import tpu as pltpu
from jax.experimental.pallas import tpu_sc as plsc

N = 10000
E = 320000
D = 128

NC = 2
NS = 16
NW = NC * NS
EPW = E // NW
BATCH = 80
NB = EPW // BATCH
N_PAD = 10240
RPZ = N_PAD // NS

_mesh = plsc.VectorSubcoreMesh(core_axis_name="c", subcore_axis_name="s")


def _deg_body(dst_hbm, ew_hbm, zero_hbm, out_hbm, dst_v, ew_v, shared_deg):
    c = lax.axis_index("c")
    s = lax.axis_index("s")
    row0 = pl.multiple_of(s * RPZ, 8)
    pltpu.sync_copy(dst_hbm.at[c, s], dst_v)
    pltpu.sync_copy(ew_hbm.at[c, s], ew_v)
    pltpu.sync_copy(zero_hbm.at[pl.ds(row0, RPZ)], shared_deg.at[pl.ds(row0, RPZ)])
    plsc.subcore_barrier()

    def body(j, carry):
        pltpu.sync_copy(ew_v.at[j], shared_deg.at[dst_v.at[j]], add=True)
        return carry

    lax.fori_loop(0, NB, body, 0)
    plsc.subcore_barrier()
    pltpu.sync_copy(shared_deg.at[pl.ds(row0, RPZ)], out_hbm.at[c, pl.ds(row0, RPZ)])


_deg_sc = pl.kernel(
    _deg_body,
    out_type=jax.ShapeDtypeStruct((NC, N_PAD), jnp.float32),
    mesh=_mesh,
    scratch_types=[
        pltpu.VMEM((NB, BATCH), jnp.int32),
        pltpu.VMEM((NB, BATCH), jnp.float32),
        pltpu.VMEM_SHARED((N_PAD,), jnp.float32),
    ],
    compiler_params=pltpu.CompilerParams(needs_layout_passes=False),
)


CHUNK_E = 2000
NCH = EPW // CHUNK_E
NB_C = CHUNK_E // BATCH


def _agg_body(h_hbm, src_hbm, dst_hbm, ew_hbm, zero_hbm, out_hbm,
              src_q, dst_q, ew_q, rows2, shared, gsems):
    c = lax.axis_index("c")
    s = lax.axis_index("s")
    row0 = pl.multiple_of(s * RPZ, 8)
    pltpu.sync_copy(zero_hbm.at[pl.ds(row0, RPZ)], shared.at[pl.ds(row0, RPZ)])
    plsc.subcore_barrier()

    def gather(j, p):
        e0 = pl.multiple_of(j * BATCH, 8)
        return pltpu.make_async_copy(
            h_hbm.at[src_q.at[pl.ds(e0, BATCH)]], rows2.at[p], gsems.at[p])

    def qbody(q, carry):
        pltpu.sync_copy(src_hbm.at[c, s, q], src_q)
        pltpu.sync_copy(dst_hbm.at[c, s, q], dst_q)
        pltpu.sync_copy(ew_hbm.at[c, s, q], ew_q)
        gather(0, 0).start()

        def jbody(j, carry2):
            p = lax.rem(j, 2)
            gather(j, p).wait()

            @pl.when(j < NB_C - 1)
            def _():
                gather(j + 1, 1 - p).start()

            @plsc.parallel_loop(0, BATCH, 1, unroll=4)
            def ebody(e):
                w = plsc.load_gather(
                    ew_q, [jnp.full((16,), j * BATCH + e, jnp.int32)])
                for k in range(D // 16):
                    sl = pl.ds(k * 16, 16)
                    rows2[p, e, sl] = rows2[p, e, sl] * w

            pltpu.sync_copy(rows2.at[p], shared.at[dst_q.at[j]], add=True)
            return carry2

        lax.fori_loop(0, NB_C, jbody, 0)
        return carry

    lax.fori_loop(0, NCH, qbody, 0)
    plsc.subcore_barrier()
    pltpu.sync_copy(shared.at[pl.ds(row0, RPZ)], out_hbm.at[c, pl.ds(row0, RPZ)])


_agg_sc = pl.kernel(
    _agg_body,
    out_type=jax.ShapeDtypeStruct((NC, N_PAD, D), jnp.float32),
    mesh=_mesh,
    scratch_types=[
        pltpu.VMEM((CHUNK_E,), jnp.int32),
        pltpu.VMEM((NB_C, BATCH), jnp.int32),
        pltpu.VMEM((CHUNK_E,), jnp.float32),
        pltpu.VMEM((2, BATCH, D), jnp.float32),
        pltpu.VMEM_SHARED((N_PAD, D), jnp.float32),
        pltpu.SemaphoreType.DMA((2,)),
    ],
    compiler_params=pltpu.CompilerParams(needs_layout_passes=False),
)


BR = 2000


def _tc1_body(d0_ref, d1_ref, x_ref, w_ref, h1p_ref, dis_ref):
    deg = d0_ref[...] + d1_ref[...] + 1.0
    dis = jnp.where(deg > 0, lax.rsqrt(deg), 0.0)
    h = jnp.dot(x_ref[...], w_ref[...], preferred_element_type=jnp.float32)
    h1p_ref[...] = dis * h
    dis_ref[...] = dis


def _tc2_body(s0_ref, s1_ref, hp_ref, dis_ref, b_ref, w_ref, out_ref):
    dis = dis_ref[...]
    z = dis * (s0_ref[...] + s1_ref[...] + hp_ref[...]) + b_ref[...]
    z = jnp.maximum(z, 0.0)
    out_ref[...] = dis * jnp.dot(z, w_ref[...], preferred_element_type=jnp.float32)


def _tc3_body(s0_ref, s1_ref, hp_ref, dis_ref, b_ref, g_ref, be_ref, out_ref):
    a = dis_ref[...] * (s0_ref[...] + s1_ref[...] + hp_ref[...]) + b_ref[...]
    m = jnp.mean(a, axis=1, keepdims=True)
    v = jnp.mean((a - m) ** 2, axis=1, keepdims=True)
    out_ref[...] = (a - m) * lax.rsqrt(v + 1e-5) * g_ref[...] + be_ref[...]


_rows_spec = pl.BlockSpec((BR, D), lambda i: (i, 0))
_col_spec = pl.BlockSpec((BR, 1), lambda i: (i, 0))
_w_spec = pl.BlockSpec((D, D), lambda i: (0, 0))
_vec_spec = pl.BlockSpec((1, D), lambda i: (0, 0))

_tc1 = pl.pallas_call(
    _tc1_body,
    grid=(N // BR,),
    in_specs=[_col_spec, _col_spec, _rows_spec, _w_spec],
    out_specs=[_rows_spec, _col_spec],
    out_shape=[
        jax.ShapeDtypeStruct((N, D), jnp.float32),
        jax.ShapeDtypeStruct((N, 1), jnp.float32),
    ],
)

_tc2 = pl.pallas_call(
    _tc2_body,
    grid=(N // BR,),
    in_specs=[_rows_spec, _rows_spec, _rows_spec, _col_spec, _vec_spec, _w_spec],
    out_specs=_rows_spec,
    out_shape=jax.ShapeDtypeStruct((N, D), jnp.float32),
)

_tc3 = pl.pallas_call(
    _tc3_body,
    grid=(N // BR,),
    in_specs=[_rows_spec, _rows_spec, _rows_spec, _col_spec, _vec_spec,
              _vec_spec, _vec_spec],
    out_specs=_rows_spec,
    out_shape=jax.ShapeDtypeStruct((N, D), jnp.float32),
)


@jax.jit
def kernel(x, edge_index, edge_weight, W1, b1, W2, b2, gamma, beta):
    src = edge_index[0].astype(jnp.int32)
    dst = edge_index[1].astype(jnp.int32)
    ew = edge_weight.astype(jnp.float32)

    src_r = src.reshape(NC, NS, NCH, CHUNK_E)
    dst_r = dst.reshape(NC, NS, NCH, NB_C, BATCH)
    ew_r = ew.reshape(NC, NS, NCH, CHUNK_E)
    ew_b = ew.reshape(NC, NS, NB, BATCH)
    dst_b = dst.reshape(NC, NS, NB, BATCH)
    zero_col = jnp.zeros((N_PAD,), jnp.float32)
    zero_rows = jnp.zeros((N_PAD, D), jnp.float32)

    deg_p = _deg_sc(dst_b, ew_b, zero_col)
    d0 = deg_p[0, :N].reshape(N, 1)
    d1 = deg_p[1, :N].reshape(N, 1)
    h1p, dis = _tc1(d0, d1, x, W1)

    s1 = _agg_sc(h1p, src_r, dst_r, ew_r, zero_rows)
    h2p = _tc2(s1[0, :N], s1[1, :N], h1p, dis, b1.reshape(1, D), W2)

    s2 = _agg_sc(h2p, src_r, dst_r, ew_r, zero_rows)
    out = _tc3(s2[0, :N], s2[1, :N], h2p, dis, b2.reshape(1, D),
               gamma.reshape(1, D), beta.reshape(1, D))
    return out


# speedup vs baseline: 26.2582x; 1.1459x over previous
import jax
import jax.numpy as jnp
from jax import lax
from jax.experimental import pallas as pl
from jax.experimental.pallas import tpu as pltpu
from jax.experimental.pallas import tpu_sc as plsc

N = 10000
E = 320000
D = 128

NC = 2
NS = 16
NW = NC * NS
EPW = E // NW
BATCH = 80
NB = EPW // BATCH
N_PAD = 10240
RPZ = N_PAD // NS

_mesh = plsc.VectorSubcoreMesh(core_axis_name="c", subcore_axis_name="s")


def _deg_body(dst_hbm, ew_hbm, zero_hbm, out_hbm, dst_v, ew_v, shared_deg):
    c = lax.axis_index("c")
    s = lax.axis_index("s")
    row0 = pl.multiple_of(s * RPZ, 8)
    pltpu.sync_copy(dst_hbm.at[c, s], dst_v)
    pltpu.sync_copy(ew_hbm.at[c, s], ew_v)
    pltpu.sync_copy(zero_hbm.at[pl.ds(row0, RPZ)], shared_deg.at[pl.ds(row0, RPZ)])
    plsc.subcore_barrier()

    def body(j, carry):
        pltpu.sync_copy(ew_v.at[j], shared_deg.at[dst_v.at[j]], add=True)
        return carry

    lax.fori_loop(0, NB, body, 0)
    plsc.subcore_barrier()
    pltpu.sync_copy(shared_deg.at[pl.ds(row0, RPZ)], out_hbm.at[c, pl.ds(row0, RPZ)])


_deg_sc = pl.kernel(
    _deg_body,
    out_type=jax.ShapeDtypeStruct((NC, N_PAD), jnp.float32),
    mesh=_mesh,
    scratch_types=[
        pltpu.VMEM((NB, BATCH), jnp.int32),
        pltpu.VMEM((NB, BATCH), jnp.float32),
        pltpu.VMEM_SHARED((N_PAD,), jnp.float32),
    ],
    compiler_params=pltpu.CompilerParams(needs_layout_passes=False),
)


CHUNK_E = 2000
NCH = EPW // CHUNK_E
NB_C = CHUNK_E // BATCH


def _agg_body(h_hbm, src_hbm, dst_hbm, ew_hbm, zero_hbm, out_hbm,
              src_q, dst_q, ew_q, rows3, shared, gsems, ssems):
    c = lax.axis_index("c")
    s = lax.axis_index("s")
    row0 = pl.multiple_of(s * RPZ, 8)
    pltpu.sync_copy(zero_hbm.at[pl.ds(row0, RPZ)], shared.at[pl.ds(row0, RPZ)])
    plsc.subcore_barrier()

    def gather(j, p):
        e0 = pl.multiple_of(j * BATCH, 8)
        return pltpu.make_async_copy(
            h_hbm.at[src_q.at[pl.ds(e0, BATCH)]], rows3.at[p], gsems.at[p])

    def scatter_start(j, p):
        pltpu.async_copy(rows3.at[p], shared.at[dst_q.at[j]], ssems.at[p],
                         add=True)

    def scatter_wait(j, p):
        pltpu.make_async_copy(
            rows3.at[p], shared.at[dst_q.at[j]], ssems.at[p]).wait()

    def qbody(q, carry):
        pltpu.sync_copy(src_hbm.at[c, s, q], src_q)
        pltpu.sync_copy(dst_hbm.at[c, s, q], dst_q)
        pltpu.sync_copy(ew_hbm.at[c, s, q], ew_q)
        gather(0, 0).start()
        gather(1, 1).start()

        def jbody(j, carry2):
            p = lax.rem(j, 3)
            gather(j, p).wait()

            @plsc.parallel_loop(0, BATCH, 1, unroll=4)
            def ebody(e):
                w = plsc.load_gather(
                    ew_q, [jnp.full((16,), j * BATCH + e, jnp.int32)])
                for k in range(D // 16):
                    sl = pl.ds(k * 16, 16)
                    rows3[p, e, sl] = rows3[p, e, sl] * w

            scatter_start(j, p)

            @pl.when(j + 2 < NB_C)
            def _():
                p2 = lax.rem(j + 2, 3)

                @pl.when(j > 0)
                def _():
                    scatter_wait(j - 1, p2)

                gather(j + 2, p2).start()

            return carry2

        lax.fori_loop(0, NB_C, jbody, 0)
        for jj in range(NB_C - 3, NB_C):
            scatter_wait(jj, jj % 3)
        return carry

    lax.fori_loop(0, NCH, qbody, 0)
    plsc.subcore_barrier()
    pltpu.sync_copy(shared.at[pl.ds(row0, RPZ)], out_hbm.at[c, pl.ds(row0, RPZ)])


_agg_sc = pl.kernel(
    _agg_body,
    out_type=jax.ShapeDtypeStruct((NC, N_PAD, D), jnp.float32),
    mesh=_mesh,
    scratch_types=[
        pltpu.VMEM((CHUNK_E,), jnp.int32),
        pltpu.VMEM((NB_C, BATCH), jnp.int32),
        pltpu.VMEM((CHUNK_E,), jnp.float32),
        pltpu.VMEM((3, BATCH, D), jnp.float32),
        pltpu.VMEM_SHARED((N_PAD, D), jnp.float32),
        pltpu.SemaphoreType.DMA((3,)),
        pltpu.SemaphoreType.DMA((3,)),
    ],
    compiler_params=pltpu.CompilerParams(needs_layout_passes=False),
)


BR = 2000


def _tc1_body(d0_ref, d1_ref, x_ref, w_ref, h1p_ref, dis_ref):
    deg = d0_ref[...] + d1_ref[...] + 1.0
    dis = jnp.where(deg > 0, lax.rsqrt(deg), 0.0)
    h = jnp.dot(x_ref[...], w_ref[...], preferred_element_type=jnp.float32)
    h1p_ref[...] = dis * h
    dis_ref[...] = dis


def _tc2_body(s0_ref, s1_ref, hp_ref, dis_ref, b_ref, w_ref, out_ref):
    dis = dis_ref[...]
    z = dis * (s0_ref[...] + s1_ref[...] + hp_ref[...]) + b_ref[...]
    z = jnp.maximum(z, 0.0)
    out_ref[...] = dis * jnp.dot(z, w_ref[...], preferred_element_type=jnp.float32)


def _tc3_body(s0_ref, s1_ref, hp_ref, dis_ref, b_ref, g_ref, be_ref, out_ref):
    a = dis_ref[...] * (s0_ref[...] + s1_ref[...] + hp_ref[...]) + b_ref[...]
    m = jnp.mean(a, axis=1, keepdims=True)
    v = jnp.mean((a - m) ** 2, axis=1, keepdims=True)
    out_ref[...] = (a - m) * lax.rsqrt(v + 1e-5) * g_ref[...] + be_ref[...]


_rows_spec = pl.BlockSpec((BR, D), lambda i: (i, 0))
_col_spec = pl.BlockSpec((BR, 1), lambda i: (i, 0))
_w_spec = pl.BlockSpec((D, D), lambda i: (0, 0))
_vec_spec = pl.BlockSpec((1, D), lambda i: (0, 0))

_tc1 = pl.pallas_call(
    _tc1_body,
    grid=(N // BR,),
    in_specs=[_col_spec, _col_spec, _rows_spec, _w_spec],
    out_specs=[_rows_spec, _col_spec],
    out_shape=[
        jax.ShapeDtypeStruct((N, D), jnp.float32),
        jax.ShapeDtypeStruct((N, 1), jnp.float32),
    ],
)

_tc2 = pl.pallas_call(
    _tc2_body,
    grid=(N // BR,),
    in_specs=[_rows_spec, _rows_spec, _rows_spec, _col_spec, _vec_spec, _w_spec],
    out_specs=_rows_spec,
    out_shape=jax.ShapeDtypeStruct((N, D), jnp.float32),
)

_tc3 = pl.pallas_call(
    _tc3_body,
    grid=(N // BR,),
    in_specs=[_rows_spec, _rows_spec, _rows_spec, _col_spec, _vec_spec,
              _vec_spec, _vec_spec],
    out_specs=_rows_spec,
    out_shape=jax.ShapeDtypeStruct((N, D), jnp.float32),
)


@jax.jit
def kernel(x, edge_index, edge_weight, W1, b1, W2, b2, gamma, beta):
    src = edge_index[0].astype(jnp.int32)
    dst = edge_index[1].astype(jnp.int32)
    ew = edge_weight.astype(jnp.float32)

    src_r = src.reshape(NC, NS, NCH, CHUNK_E)
    dst_r = dst.reshape(NC, NS, NCH, NB_C, BATCH)
    ew_r = ew.reshape(NC, NS, NCH, CHUNK_E)
    ew_b = ew.reshape(NC, NS, NB, BATCH)
    dst_b = dst.reshape(NC, NS, NB, BATCH)
    zero_col = jnp.zeros((N_PAD,), jnp.float32)
    zero_rows = jnp.zeros((N_PAD, D), jnp.float32)

    deg_p = _deg_sc(dst_b, ew_b, zero_col)
    d0 = deg_p[0, :N].reshape(N, 1)
    d1 = deg_p[1, :N].reshape(N, 1)
    h1p, dis = _tc1(d0, d1, x, W1)

    s1 = _agg_sc(h1p, src_r, dst_r, ew_r, zero_rows)
    h2p = _tc2(s1[0, :N], s1[1, :N], h1p, dis, b1.reshape(1, D), W2)

    s2 = _agg_sc(h2p, src_r, dst_r, ew_r, zero_rows)
    out = _tc3(s2[0, :N], s2[1, :N], h2p, dis, b2.reshape(1, D),
               gamma.reshape(1, D), beta.reshape(1, D))
    return out
